# baseline (speedup 1.0000x reference)
import functools

import jax
import jax.numpy as jnp
from jax import lax
from jax.experimental import pallas as pl
from jax.experimental.pallas import tpu as pltpu
from jax.experimental.pallas import tpu_sc as plsc

NUM_TOPIC = 18115
D = 64
H = 32
B, L = 16384, 50

NPAD = 18176
PKW = 48
BR = 2272

NC, NS = 2, 16
NW = NC * NS
BAGS_W = B // NW
BAGS_PER_CHUNK = 2
CHUNKS = BAGS_W // BAGS_PER_CHUNK
CPI = BAGS_PER_CHUNK * L
NBUF = 4
STAGE_ROWS = NPAD // NS
OUT_STAGE = 64


def _prep_body(tb_ref, w1_ref, b1_ref, w2_ref, b2_ref, out_ref):
    tb = tb_ref[...]
    e = jnp.tanh(
        lax.dot_general(tb, w1_ref[...], (((1,), (1,)), ((), ())),
                        preferred_element_type=jnp.float32) + b1_ref[...])
    s = jnp.exp(jnp.sum(e * w2_ref[...], axis=1, keepdims=True) + b2_ref[0, 0])
    rows = pl.program_id(0) * BR + lax.broadcasted_iota(jnp.int32, (BR, 1), 0)
    s = jnp.where(rows != 0, s, 0.0)
    num = tb * s
    den = jnp.broadcast_to(s, (BR, 16))
    zero = jnp.zeros((BR, 16), jnp.float32)

    def bf16_bits(x):
        bits = lax.bitcast_convert_type(x, jnp.int32)
        r = bits + 0x7FFF + ((bits >> 16) & 1)
        return (r >> 16) & 0xFFFF

    def word(lo, hi):
        return bf16_bits(lo) | (bf16_bits(hi) << 16)

    out_ref[...] = jnp.concatenate(
        [word(num[:, 0:16], num[:, 16:32]),
         word(num[:, 32:48], num[:, 48:64]),
         word(den, zero)], axis=1)


def _prep(table_p, W1, b1, W2, b2):
    return pl.pallas_call(
        _prep_body,
        grid=(NPAD // BR,),
        in_specs=[
            pl.BlockSpec((BR, D), lambda i: (i, 0)),
            pl.BlockSpec((H, D), lambda i: (0, 0)),
            pl.BlockSpec((1, H), lambda i: (0, 0)),
            pl.BlockSpec((1, H), lambda i: (0, 0)),
            pl.BlockSpec(memory_space=pltpu.SMEM),
        ],
        out_specs=pl.BlockSpec((BR, PKW), lambda i: (i, 0)),
        out_shape=jax.ShapeDtypeStruct((NPAD, PKW), jnp.int32),
    )(table_p, W1, b1.reshape(1, H), W2, b2.reshape(1, 1))


@functools.cache
def _make_pool():
    mesh = plsc.VectorSubcoreMesh(
        core_axis_name="c", subcore_axis_name="s", num_cores=1)
    return functools.partial(
        pl.kernel,
        mesh=mesh,
        out_type=jax.ShapeDtypeStruct((B // 2, D), jnp.float32),
        scratch_types=[
            pltpu.VMEM((CHUNKS, CPI), jnp.int32),
            pltpu.VMEM((NBUF, CPI, PKW), jnp.int32),
            pltpu.VMEM((OUT_STAGE, D), jnp.float32),
            pltpu.VMEM_SHARED((NPAD, PKW), jnp.int32),
            [pltpu.SemaphoreType.DMA] * NBUF,
        ],
        compiler_params=pltpu.CompilerParams(use_tc_tiling_on_sc=False),
    )(_pool_body)


def _bits_to_f32(w):
    return lax.bitcast_convert_type(w, jnp.float32)


def _pool_body(aug_hbm, ids_hbm, out_hbm, idx_v, rows_v, outs_v, aug_sh, sems):
    wid = lax.axis_index("s")
    pltpu.sync_copy(aug_hbm.at[pl.ds(wid * STAGE_ROWS, STAGE_ROWS)],
                    aug_sh.at[pl.ds(wid * STAGE_ROWS, STAGE_ROWS)])
    pltpu.sync_copy(ids_hbm.at[wid], idx_v)
    plsc.subcore_barrier()

    def fire(c, slot):
        pltpu.async_copy(aug_sh.at[idx_v.at[c]], rows_v.at[slot], sems[slot])

    def wait(c, slot):
        pltpu.make_async_copy(
            aug_sh.at[idx_v.at[c]], rows_v.at[slot], sems[slot]).wait()

    for s in range(NBUF):
        fire(s, s)

    def outer(i, _):
        for slot in range(NBUF):
            c = NBUF * i + slot
            wait(c, slot)
            for bg in range(BAGS_PER_CHUNK):
                bag = BAGS_PER_CHUNK * c + bg

                def rbody(r2, acc, _bg=bg, _slot=slot):
                    new = list(acc)
                    for dr in range(2):
                        row = _bg * L + 2 * r2 + dr
                        for g in range(3):
                            w = rows_v[_slot, row, pl.ds(16 * g, 16)]
                            new[2 * g] = new[2 * g] + _bits_to_f32(w << 16)
                            if g < 2:
                                new[2 * g + 1] = (new[2 * g + 1]
                                                  + _bits_to_f32(w & -65536))
                    return tuple(new)

                acc = lax.fori_loop(
                    0, L // 2, rbody,
                    tuple(jnp.zeros((16,), jnp.float32) for _ in range(5)))
                inv = 1.0 / (acc[4] + 1e-8)
                for j in range(D // 16):
                    outs_v[bag % OUT_STAGE, pl.ds(16 * j, 16)] = acc[j] * inv

            @pl.when(c + NBUF < CHUNKS)
            def _(c=c, slot=slot):
                fire(c + NBUF, slot)

            cpb = OUT_STAGE // BAGS_PER_CHUNK

            @pl.when(c % cpb == cpb - 1)
            def _(c=c):
                pltpu.sync_copy(
                    outs_v,
                    out_hbm.at[pl.ds(
                        wid * BAGS_W + (c // cpb) * OUT_STAGE, OUT_STAGE)])

        return 0

    lax.fori_loop(0, CHUNKS // NBUF, outer, 0)


def kernel(topic_ids, table, W1, b1, W2, b2):
    aug = _prep(table, W1, b1, W2, b2)
    ids = topic_ids.astype(jnp.int32).reshape(NW, CHUNKS, CPI)
    pool = _make_pool()
    return jnp.concatenate([pool(aug, ids[:NS]), pool(aug, ids[NS:])], axis=0)


# speedup vs baseline: 1.4623x; 1.4623x over previous
import functools

import jax
import jax.numpy as jnp
from jax import lax
from jax.experimental import pallas as pl
from jax.experimental.pallas import tpu as pltpu
from jax.experimental.pallas import tpu_sc as plsc

NUM_TOPIC = 18115
D = 64
H = 32
B, L = 16384, 50

NPAD = 18176
PKW = 48
BR = 2272

NC, NS = 2, 16
NW = NC * NS
BAGS_W = B // NW
BAGS_PER_CHUNK = 2
CHUNKS = BAGS_W // BAGS_PER_CHUNK
CPI = BAGS_PER_CHUNK * L
NBUF = 4
STAGE_ROWS = NPAD // NS
OUT_STAGE = 64


def _prep_body(tb_ref, w1_ref, b1_ref, w2_ref, b2_ref, out_ref):
    tb = tb_ref[...]
    e = jnp.tanh(
        lax.dot_general(tb, w1_ref[...], (((1,), (1,)), ((), ())),
                        preferred_element_type=jnp.float32) + b1_ref[...])
    s = jnp.exp(jnp.sum(e * w2_ref[...], axis=1, keepdims=True) + b2_ref[0, 0])
    rows = pl.program_id(0) * BR + lax.broadcasted_iota(jnp.int32, (BR, 1), 0)
    s = jnp.where(rows != 0, s, 0.0)
    num = tb * s
    den = jnp.broadcast_to(s, (BR, 16))
    zero = jnp.zeros((BR, 16), jnp.float32)

    def bf16_bits(x):
        bits = lax.bitcast_convert_type(x, jnp.int32)
        r = bits + 0x7FFF + ((bits >> 16) & 1)
        return (r >> 16) & 0xFFFF

    def word(lo, hi):
        return bf16_bits(lo) | (bf16_bits(hi) << 16)

    out_ref[...] = jnp.concatenate(
        [word(num[:, 0:16], num[:, 16:32]),
         word(num[:, 32:48], num[:, 48:64]),
         word(den, zero)], axis=1)


def _prep(table_p, W1, b1, W2, b2):
    return pl.pallas_call(
        _prep_body,
        grid=(NPAD // BR,),
        in_specs=[
            pl.BlockSpec((BR, D), lambda i: (i, 0)),
            pl.BlockSpec((H, D), lambda i: (0, 0)),
            pl.BlockSpec((1, H), lambda i: (0, 0)),
            pl.BlockSpec((1, H), lambda i: (0, 0)),
            pl.BlockSpec(memory_space=pltpu.SMEM),
        ],
        out_specs=pl.BlockSpec((BR, PKW), lambda i: (i, 0)),
        out_shape=jax.ShapeDtypeStruct((NPAD, PKW), jnp.int32),
    )(table_p, W1, b1.reshape(1, H), W2, b2.reshape(1, 1))


@functools.cache
def _make_pool():
    mesh = plsc.VectorSubcoreMesh(core_axis_name="c", subcore_axis_name="s")
    return functools.partial(
        pl.kernel,
        mesh=mesh,
        out_type=jax.ShapeDtypeStruct((B, D), jnp.float32),
        scratch_types=[
            pltpu.VMEM((CHUNKS, CPI), jnp.int32),
            pltpu.VMEM((NBUF, CPI, PKW), jnp.int32),
            pltpu.VMEM((OUT_STAGE, D), jnp.float32),
            pltpu.VMEM_SHARED((NPAD, PKW), jnp.int32),
            [pltpu.SemaphoreType.DMA] * NBUF,
        ],
        compiler_params=pltpu.CompilerParams(use_tc_tiling_on_sc=False),
    )(_pool_body)


def _bits_to_f32(w):
    return lax.bitcast_convert_type(w, jnp.float32)


def _pool_body(aug_hbm, ids_hbm, out_hbm, idx_v, rows_v, outs_v, aug_sh, sems):
    sub = lax.axis_index("s")
    wid = sub * NC + lax.axis_index("c")
    pltpu.sync_copy(aug_hbm.at[pl.ds(sub * STAGE_ROWS, STAGE_ROWS)],
                    aug_sh.at[pl.ds(sub * STAGE_ROWS, STAGE_ROWS)])
    pltpu.sync_copy(ids_hbm.at[wid], idx_v)
    plsc.subcore_barrier()

    def fire(c, slot):
        pltpu.async_copy(aug_sh.at[idx_v.at[c]], rows_v.at[slot], sems[slot])

    def wait(c, slot):
        pltpu.make_async_copy(
            aug_sh.at[idx_v.at[c]], rows_v.at[slot], sems[slot]).wait()

    for s in range(NBUF):
        fire(s, s)

    def outer(i, _):
        for slot in range(NBUF):
            c = NBUF * i + slot
            wait(c, slot)
            for bg in range(BAGS_PER_CHUNK):
                bag = BAGS_PER_CHUNK * c + bg

                def rbody(r2, acc, _bg=bg, _slot=slot):
                    new = list(acc)
                    for dr in range(2):
                        row = _bg * L + 2 * r2 + dr
                        for g in range(3):
                            w = rows_v[_slot, row, pl.ds(16 * g, 16)]
                            new[2 * g] = new[2 * g] + _bits_to_f32(w << 16)
                            if g < 2:
                                new[2 * g + 1] = (new[2 * g + 1]
                                                  + _bits_to_f32(w & -65536))
                    return tuple(new)

                acc = lax.fori_loop(
                    0, L // 2, rbody,
                    tuple(jnp.zeros((16,), jnp.float32) for _ in range(5)))
                inv = 1.0 / (acc[4] + 1e-8)
                for j in range(D // 16):
                    outs_v[bag % OUT_STAGE, pl.ds(16 * j, 16)] = acc[j] * inv

            @pl.when(c + NBUF < CHUNKS)
            def _(c=c, slot=slot):
                fire(c + NBUF, slot)

            cpb = OUT_STAGE // BAGS_PER_CHUNK

            @pl.when(c % cpb == cpb - 1)
            def _(c=c):
                pltpu.sync_copy(
                    outs_v,
                    out_hbm.at[pl.ds(
                        wid * BAGS_W + (c // cpb) * OUT_STAGE, OUT_STAGE)])

        return 0

    lax.fori_loop(0, CHUNKS // NBUF, outer, 0)


def kernel(topic_ids, table, W1, b1, W2, b2):
    aug = _prep(table, W1, b1, W2, b2)
    ids = topic_ids.astype(jnp.int32).reshape(NW, CHUNKS, CPI)
    return _make_pool()(aug, ids)
